# baseline (speedup 1.0000x reference)
import jax
import jax.numpy as jnp
from jax import lax
from jax.experimental import pallas as pl
from jax.experimental.pallas import tpu as pltpu
from jax.experimental.pallas import tpu_sc as plsc

_N, _B, _D = 2048, 4, 128
_BN = 128

_NW = 32
_RPW = (_N * _B) // _NW
_CH = 128
_NCH = _RPW // _CH


def _levels_body(inc_ref, out_ref):
    counts = jnp.sum(inc_ref[...], axis=-1)
    out_ref[...] = counts.T + 1


def _compute_levels_t(node_incidences):
    return pl.pallas_call(
        _levels_body,
        grid=(_N // _BN,),
        in_specs=[pl.BlockSpec((_B, _BN, _N), lambda n: (0, n, 0))],
        out_specs=pl.BlockSpec((_BN, _B), lambda n: (n, 0)),
        out_shape=jax.ShapeDtypeStruct((_N, _B), jnp.int32),
    )(node_incidences)


def _gather_add_body(x_hbm, idx_hbm, tab_hbm, out_hbm,
                     idx_v, gat_v, x_v, sem_g, sem_x):
    wid = lax.axis_index("s") * 2 + lax.axis_index("c")
    row0 = wid * _RPW
    pltpu.sync_copy(idx_hbm.at[pl.ds(wid * _NCH, _NCH)], idx_v)
    copies = []
    for k in range(_NCH):
        copies.append(pltpu.async_copy(tab_hbm.at[idx_v.at[k]], gat_v.at[k], sem_g))
        copies.append(pltpu.async_copy(
            x_hbm.at[pl.ds(row0 + k * _CH, _CH)], x_v.at[k], sem_x))
    for cp in copies:
        cp.wait()

    def row(r, carry):
        for k in range(_NCH):
            for c in range(_D // 16):
                s = pl.ds(c * 16, 16)
                gat_v[k, r, s] = gat_v[k, r, s] + x_v[k, r, s]
        return carry

    lax.fori_loop(0, _CH, row, 0)
    for k in range(_NCH):
        pltpu.sync_copy(gat_v.at[k], out_hbm.at[pl.ds(row0 + k * _CH, _CH)])


def _gather_add(x_flat, idx, table):
    mesh = plsc.VectorSubcoreMesh(core_axis_name="c", subcore_axis_name="s")
    f = pl.kernel(
        _gather_add_body,
        mesh=mesh,
        out_type=jax.ShapeDtypeStruct((_N * _B, _D), jnp.float32),
        scratch_types=[
            pltpu.VMEM((_NCH, _CH), jnp.int32),
            pltpu.VMEM((_NCH, _CH, _D), jnp.float32),
            pltpu.VMEM((_NCH, _CH, _D), jnp.float32),
            pltpu.SemaphoreType.DMA,
            pltpu.SemaphoreType.DMA,
        ],
    )
    return f(x_flat, idx, table)


def kernel(x, node_incidences, pos_embedding):
    levels_t = _compute_levels_t(node_incidences)
    idx = levels_t.reshape(_N * _B // _CH, _CH)
    x_flat = x.reshape(_N * _B, _D)
    out = _gather_add(x_flat, idx, pos_embedding)
    return out.reshape(_N, _B, _D)
